# baseline (speedup 1.0000x reference)
import functools

import jax
import jax.numpy as jnp
from jax import lax
from jax.experimental import pallas as pl
from jax.experimental.pallas import tpu as pltpu
from jax.experimental.pallas import tpu_sc as plsc

_R = 8


def _build_sc_kernel(B, L, D, NW):
    RW = L // NW
    NCH = RW // _R
    NPAIR = NCH // 2
    STEPS = NCH * B
    CADD = D // 16

    mesh = plsc.VectorSubcoreMesh(core_axis_name="c", subcore_axis_name="s")

    @functools.partial(
        pl.kernel,
        out_type=jax.ShapeDtypeStruct((B, L, D), jnp.float32),
        mesh=mesh,
        scratch_types=(
            [pltpu.VMEM((_R, D), jnp.float32) for _ in range(4)]
            + [pltpu.VMEM((_R, D), jnp.float32) for _ in range(2)]
            + [pltpu.VMEM((2 * _R,), jnp.int32) for _ in range(2)]
            + [pltpu.SemaphoreType.DMA for _ in range(12)]
        ),
    )
    def body(x_hbm, emb_hbm, p_hbm, out_hbm,
             xb0, xb1, xb2, xb3, eb0, eb1, ib0, ib1,
             si0, si1, si2, si3, so0, so1, so2, so3, se0, se1, sp0, sp1):
        xbufs = (xb0, xb1, xb2, xb3)
        ebufs = (eb0, eb1)
        ibufs = (ib0, ib1)
        sin = (si0, si1, si2, si3)
        sout = (so0, so1, so2, so3)
        semb = (se0, se1)
        sidx = (sp0, sp1)

        wid = lax.axis_index("s") * 2 + lax.axis_index("c")
        base = wid * RW

        def rows(c):
            return base + c * _R

        def start_in(s):
            c, b = divmod(s, B)
            return pltpu.async_copy(
                x_hbm.at[b, pl.ds(rows(c), _R)], xbufs[s % 4], sin[s % 4])

        def start_idx(p):
            return pltpu.async_copy(
                p_hbm.at[pl.ds(base + p * 2 * _R, 2 * _R)],
                ibufs[p % 2], sidx[p % 2])

        def start_gather(c):
            p = c // 2
            half = (c % 2) * _R
            return pltpu.async_copy(
                emb_hbm.at[ibufs[p % 2].at[pl.ds(half, _R)]],
                ebufs[c % 2], semb[c % 2])

        d_in = [None] * STEPS
        d_out = [None] * STEPS
        d_emb = [None] * NCH
        d_idx = [None] * NPAIR
        d_idx[0] = start_idx(0)
        for s in range(min(3, STEPS)):
            d_in[s] = start_in(s)
        d_idx[0].wait()
        d_emb[0] = start_gather(0)
        if NCH > 1:
            d_emb[1] = start_gather(1)
        if NPAIR > 1:
            d_idx[1] = start_idx(1)

        for s in range(STEPS):
            c, b = divmod(s, B)
            buf = s % 4
            xr = xbufs[buf]
            er = ebufs[c % 2]

            if b == 0:
                d_emb[c].wait()
                if c % 2 == 1 and c // 2 + 2 < NPAIR:
                    d_idx[c // 2 + 2] = start_idx(c // 2 + 2)
            d_in[s].wait()

            def add_body(k, _, xr=xr, er=er):
                r = k // CADD
                j = (k - r * CADD) * 16
                xr[r, pl.ds(j, 16)] = xr[r, pl.ds(j, 16)] + er[r, pl.ds(j, 16)]
                return 0

            lax.fori_loop(0, _R * CADD, add_body, 0)

            d_out[s] = pltpu.async_copy(
                xr, out_hbm.at[b, pl.ds(rows(c), _R)], sout[buf])

            if s + 3 < STEPS:
                if s >= 1:
                    d_out[s - 1].wait()
                d_in[s + 3] = start_in(s + 3)

            if b == B - 1 and c + 2 < NCH:
                if (c + 2) % 2 == 0:
                    d_idx[(c + 2) // 2].wait()
                d_emb[c + 2] = start_gather(c + 2)

        for s in range(max(0, STEPS - 4), STEPS):
            d_out[s].wait()

    return body


def kernel(X, emb, offset):
    B, L, D = X.shape
    V = emb.shape[0]
    NW = 32
    assert L % (NW * 2 * _R) == 0 and D % 16 == 0
    P = jnp.clip(jnp.arange(L, dtype=jnp.int32) + offset, 0, V - 1)
    P = P.astype(jnp.int32)
    sc_add = _build_sc_kernel(B, L, D, NW)
    return sc_add(X, emb, P)


# speedup vs baseline: 2.4907x; 2.4907x over previous
import functools

import jax
import jax.numpy as jnp
from jax import lax
from jax.experimental import pallas as pl
from jax.experimental.pallas import tpu as pltpu
from jax.experimental.pallas import tpu_sc as plsc

_R = 8


def _build_sc_kernel(B, L, D, NW):
    RW = L // NW
    NCH = RW // _R
    NPAIR = NCH // 2
    STEPS = NCH * B
    CADD = D // 16

    mesh = plsc.VectorSubcoreMesh(core_axis_name="c", subcore_axis_name="s")

    @functools.partial(
        pl.kernel,
        out_type=jax.ShapeDtypeStruct((B, L, D), jnp.float32),
        mesh=mesh,
        scratch_types=(
            [pltpu.VMEM((_R, D), jnp.float32) for _ in range(4)]
            + [pltpu.VMEM((_R, D), jnp.float32) for _ in range(2)]
            + [pltpu.VMEM((2 * _R,), jnp.int32) for _ in range(2)]
            + [pltpu.SemaphoreType.DMA for _ in range(12)]
        ),
    )
    def body(x_hbm, emb_hbm, p_hbm, out_hbm,
             xb0, xb1, xb2, xb3, eb0, eb1, ib0, ib1,
             si0, si1, si2, si3, so0, so1, so2, so3, se0, se1, sp0, sp1):
        xbufs = (xb0, xb1, xb2, xb3)
        ebufs = (eb0, eb1)
        ibufs = (ib0, ib1)
        sin = (si0, si1, si2, si3)
        sout = (so0, so1, so2, so3)
        semb = (se0, se1)
        sidx = (sp0, sp1)

        wid = lax.axis_index("s") * 2 + lax.axis_index("c")
        base = wid * RW

        def rows(c):
            return base + c * _R

        def start_in(s):
            c, b = divmod(s, B)
            return pltpu.async_copy(
                x_hbm.at[b, pl.ds(rows(c), _R)], xbufs[s % 4], sin[s % 4])

        def start_idx(p):
            return pltpu.async_copy(
                p_hbm.at[pl.ds(base + p * 2 * _R, 2 * _R)],
                ibufs[p % 2], sidx[p % 2])

        def start_gather(c):
            p = c // 2
            half = (c % 2) * _R
            return pltpu.async_copy(
                emb_hbm.at[ibufs[p % 2].at[pl.ds(half, _R)]],
                ebufs[c % 2], semb[c % 2])

        d_in = [None] * STEPS
        d_out = [None] * STEPS
        d_emb = [None] * NCH
        d_idx = [None] * NPAIR
        d_idx[0] = start_idx(0)
        for s in range(min(3, STEPS)):
            d_in[s] = start_in(s)
        d_idx[0].wait()
        d_emb[0] = start_gather(0)
        if NCH > 1:
            d_emb[1] = start_gather(1)
        if NPAIR > 1:
            d_idx[1] = start_idx(1)

        for s in range(STEPS):
            c, b = divmod(s, B)
            buf = s % 4
            xr = xbufs[buf]
            er = ebufs[c % 2]

            if b == 0:
                d_emb[c].wait()
                if c % 2 == 1 and c // 2 + 2 < NPAIR:
                    d_idx[c // 2 + 2] = start_idx(c // 2 + 2)
            d_in[s].wait()

            @plsc.parallel_loop(0, _R * CADD, 1, unroll=8)
            def add_body(k, xr=xr, er=er):
                r = k // CADD
                j = (k - r * CADD) * 16
                xr[r, pl.ds(j, 16)] = xr[r, pl.ds(j, 16)] + er[r, pl.ds(j, 16)]

            d_out[s] = pltpu.async_copy(
                xr, out_hbm.at[b, pl.ds(rows(c), _R)], sout[buf])

            if s + 3 < STEPS:
                if s >= 1:
                    d_out[s - 1].wait()
                d_in[s + 3] = start_in(s + 3)

            if b == B - 1 and c + 2 < NCH:
                if (c + 2) % 2 == 0:
                    d_idx[(c + 2) // 2].wait()
                d_emb[c + 2] = start_gather(c + 2)

        for s in range(max(0, STEPS - 4), STEPS):
            d_out[s].wait()

    return body


def kernel(X, emb, offset):
    B, L, D = X.shape
    V = emb.shape[0]
    NW = 32
    assert L % (NW * 2 * _R) == 0 and D % 16 == 0
    P = jnp.clip(jnp.arange(L, dtype=jnp.int32) + offset, 0, V - 1)
    P = P.astype(jnp.int32)
    sc_add = _build_sc_kernel(B, L, D, NW)
    return sc_add(X, emb, P)


# speedup vs baseline: 2.4922x; 1.0006x over previous
import functools

import jax
import jax.numpy as jnp
from jax import lax
from jax.experimental import pallas as pl
from jax.experimental.pallas import tpu as pltpu
from jax.experimental.pallas import tpu_sc as plsc

_R = 8


def _build_sc_kernel(B, L, D, NW):
    RW = L // NW
    NCH = RW // _R
    NPAIR = NCH // 2
    STEPS = NCH * B
    CADD = D // 16

    mesh = plsc.VectorSubcoreMesh(core_axis_name="c", subcore_axis_name="s")

    @functools.partial(
        pl.kernel,
        out_type=jax.ShapeDtypeStruct((B, L, D), jnp.float32),
        mesh=mesh,
        scratch_types=(
            [pltpu.VMEM((_R, D), jnp.float32) for _ in range(4)]
            + [pltpu.VMEM((_R, D), jnp.float32) for _ in range(2)]
            + [pltpu.VMEM((2 * _R,), jnp.int32) for _ in range(2)]
            + [pltpu.SemaphoreType.DMA for _ in range(12)]
        ),
    )
    def body(x_hbm, emb_hbm, p_hbm, out_hbm,
             xb0, xb1, xb2, xb3, eb0, eb1, ib0, ib1,
             si0, si1, si2, si3, so0, so1, so2, so3, se0, se1, sp0, sp1):
        xbufs = (xb0, xb1, xb2, xb3)
        ebufs = (eb0, eb1)
        ibufs = (ib0, ib1)
        sin = (si0, si1, si2, si3)
        sout = (so0, so1, so2, so3)
        semb = (se0, se1)
        sidx = (sp0, sp1)

        wid = lax.axis_index("s") * 2 + lax.axis_index("c")
        base = wid * RW

        def rows(c):
            return base + c * _R

        def start_in(s):
            c, b = divmod(s, B)
            return pltpu.async_copy(
                x_hbm.at[b, pl.ds(rows(c), _R)], xbufs[s % 4], sin[s % 4])

        def start_idx(p):
            return pltpu.async_copy(
                p_hbm.at[pl.ds(base + p * 2 * _R, 2 * _R)],
                ibufs[p % 2], sidx[p % 2])

        def start_gather(c):
            p = c // 2
            half = (c % 2) * _R
            return pltpu.async_copy(
                emb_hbm.at[ibufs[p % 2].at[pl.ds(half, _R)]],
                ebufs[c % 2], semb[c % 2])

        d_in = [None] * STEPS
        d_out = [None] * STEPS
        d_emb = [None] * NCH
        d_idx = [None] * NPAIR
        d_idx[0] = start_idx(0)
        for s in range(min(3, STEPS)):
            d_in[s] = start_in(s)
        d_idx[0].wait()
        d_emb[0] = start_gather(0)
        if NCH > 1:
            d_emb[1] = start_gather(1)
        if NPAIR > 1:
            d_idx[1] = start_idx(1)

        for s in range(STEPS):
            c, b = divmod(s, B)
            buf = s % 4
            xr = xbufs[buf]
            er = ebufs[c % 2]

            if b == 0:
                d_emb[c].wait()
                if c % 2 == 1 and c // 2 + 2 < NPAIR:
                    d_idx[c // 2 + 2] = start_idx(c // 2 + 2)
            d_in[s].wait()

            @plsc.parallel_loop(0, _R * CADD, 1, unroll=8)
            def add_body(k, xr=xr, er=er):
                r = k // CADD
                j = (k - r * CADD) * 16
                plsc.addupdate(xr.at[r, pl.ds(j, 16)], er[r, pl.ds(j, 16)])

            d_out[s] = pltpu.async_copy(
                xr, out_hbm.at[b, pl.ds(rows(c), _R)], sout[buf])

            if s + 3 < STEPS:
                if s >= 1:
                    d_out[s - 1].wait()
                d_in[s + 3] = start_in(s + 3)

            if b == B - 1 and c + 2 < NCH:
                if (c + 2) % 2 == 0:
                    d_idx[(c + 2) // 2].wait()
                d_emb[c + 2] = start_gather(c + 2)

        for s in range(max(0, STEPS - 4), STEPS):
            d_out[s].wait()

    return body


def kernel(X, emb, offset):
    B, L, D = X.shape
    V = emb.shape[0]
    NW = 32
    assert L % (NW * 2 * _R) == 0 and D % 16 == 0
    P = jnp.clip(jnp.arange(L, dtype=jnp.int32) + offset, 0, V - 1)
    P = P.astype(jnp.int32)
    sc_add = _build_sc_kernel(B, L, D, NW)
    return sc_add(X, emb, P)


# speedup vs baseline: 2.4985x; 1.0025x over previous
import functools

import jax
import jax.numpy as jnp
from jax import lax
from jax.experimental import pallas as pl
from jax.experimental.pallas import tpu as pltpu
from jax.experimental.pallas import tpu_sc as plsc

_R = 8


def _build_sc_kernel(B, L, D, NW):
    RW = L // NW
    NCH = RW // _R
    NPAIR = NCH // 2
    STEPS = NCH * B
    CADD = D // 16

    mesh = plsc.VectorSubcoreMesh(core_axis_name="c", subcore_axis_name="s")

    @functools.partial(
        pl.kernel,
        out_type=jax.ShapeDtypeStruct((B, L, D), jnp.float32),
        mesh=mesh,
        scratch_types=(
            [pltpu.VMEM((_R, D), jnp.float32) for _ in range(5)]
            + [pltpu.VMEM((_R, D), jnp.float32) for _ in range(2)]
            + [pltpu.VMEM((2 * _R,), jnp.int32) for _ in range(2)]
            + [pltpu.SemaphoreType.DMA for _ in range(14)]
        ),
    )
    def body(x_hbm, emb_hbm, p_hbm, out_hbm,
             xb0, xb1, xb2, xb3, xb4, eb0, eb1, ib0, ib1,
             si0, si1, si2, si3, si4, so0, so1, so2, so3, so4,
             se0, se1, sp0, sp1):
        xbufs = (xb0, xb1, xb2, xb3, xb4)
        ebufs = (eb0, eb1)
        ibufs = (ib0, ib1)
        sin = (si0, si1, si2, si3, si4)
        sout = (so0, so1, so2, so3, so4)
        semb = (se0, se1)
        sidx = (sp0, sp1)

        wid = lax.axis_index("s") * 2 + lax.axis_index("c")
        base = wid * RW

        def rows(c):
            return base + c * _R

        def start_in(s):
            c, b = divmod(s, B)
            return pltpu.async_copy(
                x_hbm.at[b, pl.ds(rows(c), _R)], xbufs[s % 5], sin[s % 5])

        def start_idx(p):
            return pltpu.async_copy(
                p_hbm.at[pl.ds(base + p * 2 * _R, 2 * _R)],
                ibufs[p % 2], sidx[p % 2])

        def start_gather(c):
            p = c // 2
            half = (c % 2) * _R
            return pltpu.async_copy(
                emb_hbm.at[ibufs[p % 2].at[pl.ds(half, _R)]],
                ebufs[c % 2], semb[c % 2])

        d_in = [None] * STEPS
        d_out = [None] * STEPS
        d_emb = [None] * NCH
        d_idx = [None] * NPAIR
        d_idx[0] = start_idx(0)
        for s in range(min(4, STEPS)):
            d_in[s] = start_in(s)
        d_idx[0].wait()
        d_emb[0] = start_gather(0)
        if NCH > 1:
            d_emb[1] = start_gather(1)
        if NPAIR > 1:
            d_idx[1] = start_idx(1)

        for s in range(STEPS):
            c, b = divmod(s, B)
            buf = s % 5
            xr = xbufs[buf]
            er = ebufs[c % 2]

            if b == 0:
                d_emb[c].wait()
                if c % 2 == 1 and c // 2 + 2 < NPAIR:
                    d_idx[c // 2 + 2] = start_idx(c // 2 + 2)
            d_in[s].wait()

            @plsc.parallel_loop(0, _R * CADD, 1, unroll=8)
            def add_body(k, xr=xr, er=er):
                r = k // CADD
                j = (k - r * CADD) * 16
                plsc.addupdate(xr.at[r, pl.ds(j, 16)], er[r, pl.ds(j, 16)])

            d_out[s] = pltpu.async_copy(
                xr, out_hbm.at[b, pl.ds(rows(c), _R)], sout[buf])

            if s + 4 < STEPS:
                if s >= 1:
                    d_out[s - 1].wait()
                d_in[s + 4] = start_in(s + 4)

            if b == B - 1 and c + 2 < NCH:
                if (c + 2) % 2 == 0:
                    d_idx[(c + 2) // 2].wait()
                d_emb[c + 2] = start_gather(c + 2)

        for s in range(max(0, STEPS - 5), STEPS):
            d_out[s].wait()

    return body


def kernel(X, emb, offset):
    B, L, D = X.shape
    V = emb.shape[0]
    NW = 32
    assert L % (NW * 2 * _R) == 0 and D % 16 == 0
    P = jnp.clip(jnp.arange(L, dtype=jnp.int32) + offset, 0, V - 1)
    P = P.astype(jnp.int32)
    sc_add = _build_sc_kernel(B, L, D, NW)
    return sc_add(X, emb, P)
